# baseline (speedup 1.0000x reference)
import functools

import jax
import jax.numpy as jnp
from jax import lax
from jax.experimental import pallas as pl
from jax.experimental.pallas import tpu as pltpu
from jax.experimental.pallas import tpu_sc as plsc

BATCH = 16384
FEAT_DIM = 64
OUT_DIM = 128
HID = 128

_GATHER_CHUNK = 128


@functools.cache
def _make_sc_gather(batch, feat_dim):
    info = plsc.get_sparse_core_info()
    nw = info.num_cores * info.num_subcores
    bpw = batch // nw
    nchunks = bpw // _GATHER_CHUNK
    mesh = plsc.VectorSubcoreMesh(core_axis_name="c", subcore_axis_name="s")

    @functools.partial(
        pl.kernel,
        mesh=mesh,
        out_type=jax.ShapeDtypeStruct((batch, feat_dim), jnp.float32),
        scratch_types=[
            pltpu.VMEM((bpw,), jnp.int32),
            pltpu.VMEM((bpw, feat_dim), jnp.float32),
            pltpu.SemaphoreType.DMA,
        ],
        compiler_params=pltpu.CompilerParams(use_tc_tiling_on_sc=False),
    )
    def gather(idx_hbm, table_hbm, out_hbm, idx_v, rows_v, sem):
        wid = lax.axis_index("s") * info.num_cores + lax.axis_index("c")
        base = wid * bpw
        pltpu.sync_copy(idx_hbm.at[pl.ds(base, bpw)], idx_v)
        copies = []
        for j in range(nchunks):
            copies.append(
                pltpu.async_copy(
                    table_hbm.at[idx_v.at[pl.ds(j * _GATHER_CHUNK, _GATHER_CHUNK)]],
                    rows_v.at[pl.ds(j * _GATHER_CHUNK, _GATHER_CHUNK)],
                    sem,
                )
            )
        for c in copies:
            c.wait()
        pltpu.sync_copy(rows_v, out_hbm.at[pl.ds(base, bpw)])

    return gather


def _layernorm(x, eps=1e-5):
    mu = jnp.mean(x, axis=-1, keepdims=True)
    var = jnp.mean((x - mu) ** 2, axis=-1, keepdims=True)
    return (x - mu) / jnp.sqrt(var + eps)


def _gelu_exact(x):
    return 0.5 * x * (1.0 + lax.erf(x * 0.7071067811865476))


def _mlp_body(emb_ref, w1_ref, b1_ref, w2_ref, b2_ref, w3_ref, b3_ref, out_ref):
    x = emb_ref[...]
    h = jnp.dot(x, w1_ref[...], preferred_element_type=jnp.float32) + b1_ref[...]
    h = _gelu_exact(_layernorm(h))
    h = jnp.dot(h, w2_ref[...], preferred_element_type=jnp.float32) + b2_ref[...]
    h = _gelu_exact(_layernorm(h))
    h = jnp.dot(h, w3_ref[...], preferred_element_type=jnp.float32) + b3_ref[...]
    out_ref[...] = _gelu_exact(h)


def _tc_mlp(emb, w1, b1, w2, b2, w3, b3, blk=2048, interpret=False):
    batch = emb.shape[0]
    grid = (batch // blk,)
    rep2 = lambda i: (0, 0)
    return pl.pallas_call(
        _mlp_body,
        grid=grid,
        in_specs=[
            pl.BlockSpec((blk, emb.shape[1]), lambda i: (i, 0)),
            pl.BlockSpec(w1.shape, rep2),
            pl.BlockSpec(b1.shape, rep2),
            pl.BlockSpec(w2.shape, rep2),
            pl.BlockSpec(b2.shape, rep2),
            pl.BlockSpec(w3.shape, rep2),
            pl.BlockSpec(b3.shape, rep2),
        ],
        out_specs=pl.BlockSpec((blk, w3.shape[1]), lambda i: (i, 0)),
        out_shape=jax.ShapeDtypeStruct((batch, w3.shape[1]), jnp.float32),
        interpret=interpret,
    )(emb, w1, b1, w2, b2, w3, b3)


def kernel(user_ids, table, W1, b1, W2, b2, W3, b3):
    idx = user_ids.astype(jnp.int32)
    emb = _make_sc_gather(BATCH, FEAT_DIM)(idx, table)
    return _tc_mlp(
        emb,
        W1,
        b1.reshape(1, -1),
        W2,
        b2.reshape(1, -1),
        W3,
        b3.reshape(1, -1),
    )
